# baseline (speedup 1.0000x reference)
import functools

import jax
import jax.numpy as jnp
from jax import lax
from jax.experimental import pallas as pl
from jax.experimental.pallas import tpu as pltpu
from jax.experimental.pallas import tpu_sc as plsc

D = 64
EPS = 1e-5
L = 16
NC = 2
NS = 16
NW = NC * NS
CH = 128
NB = 4


def _rsqrt_vec(x):
    i = lax.bitcast_convert_type(x, jnp.int32)
    i = jnp.int32(0x5F375A86) - lax.shift_right_logical(i, 1)
    y = lax.bitcast_convert_type(i, jnp.float32)
    for _ in range(3):
        y = y * (jnp.float32(1.5) - jnp.float32(0.5) * x * y * y)
    return y


def _make_kernel(n_rows):
    rows_per_worker = n_rows // NW
    n_chunks = rows_per_worker // CH
    mesh = plsc.VectorSubcoreMesh(
        core_axis_name="c", subcore_axis_name="s", num_cores=NC, num_subcores=NS
    )

    @functools.partial(
        pl.kernel,
        out_type=jax.ShapeDtypeStruct((n_rows, D), jnp.float32),
        mesh=mesh,
        scratch_types=[
            pltpu.VMEM((rows_per_worker,), jnp.int32),
            pltpu.VMEM((NB, CH, D), jnp.float32),
            pltpu.VMEM((2, D), jnp.float32),
        ]
        + [pltpu.SemaphoreType.DMA] * (2 * NB),
        compiler_params=pltpu.CompilerParams(
            use_tc_tiling_on_sc=False, needs_layout_passes=False
        ),
    )
    def k(x_hbm, table_hbm, gb_hbm, out_hbm, idx_all, rows, gb_v, *sems):
        gsem = sems[:NB]
        osem = sems[NB:]
        wid = lax.axis_index("s") * NC + lax.axis_index("c")
        base = wid * rows_per_worker
        pltpu.sync_copy(x_hbm.at[pl.ds(base, rows_per_worker)], idx_all)
        pltpu.sync_copy(gb_hbm, gb_v)

        lane = lax.iota(jnp.int32, L)
        zvec = lax.bitwise_xor(lane, lane)
        gvecs = [gb_v[0, pl.ds(i * L, L)] for i in range(D // L)]
        bvecs = [gb_v[1, pl.ds(i * L, L)] for i in range(D // L)]

        def start_gather(b, j):
            pltpu.async_copy(
                table_hbm.at[idx_all.at[pl.ds(j * CH, CH)]], rows.at[b], gsem[b]
            )

        def wait_gather(b):
            pltpu.make_async_copy(
                table_hbm.at[pl.ds(0, CH)], rows.at[b], gsem[b]
            ).wait()

        def start_out(b, j):
            pltpu.async_copy(
                rows.at[b], out_hbm.at[pl.ds(base + j * CH, CH)], osem[b]
            )

        def wait_out(b):
            pltpu.make_async_copy(
                rows.at[b], out_hbm.at[pl.ds(0, CH)], osem[b]
            ).wait()

        def compute(rows_ref):
            def grp_body(g2, carry):
                row_ids = g2 * L + lane
                fzero = lax.convert_element_type(zvec, jnp.float32)
                acc = [fzero] * 4
                acc2 = [fzero] * 4
                for c in range(D):
                    col = zvec + c
                    v = plsc.load_gather(rows_ref, [row_ids, col])
                    r = c % 4
                    acc[r] = acc[r] + v
                    acc2[r] = acc2[r] + v * v
                s = (acc[0] + acc[1]) + (acc[2] + acc[3])
                q = (acc2[0] + acc2[1]) + (acc2[2] + acc2[3])
                mean = s * jnp.float32(1.0 / D)
                var = q * jnp.float32(1.0 / D) - mean * mean
                kk = _rsqrt_vec(var + jnp.float32(EPS))
                off = mean * kk
                for c in range(D):
                    col = zvec + c
                    v = plsc.load_gather(rows_ref, [row_ids, col])
                    plsc.store_scatter(rows_ref, [row_ids, col], v * kk - off)
                return carry

            lax.fori_loop(0, CH // L, grp_body, 0)

            def row_body(r, carry):
                for i in range(D // L):
                    sl = pl.ds(i * L, L)
                    rows_ref[r, sl] = rows_ref[r, sl] * gvecs[i] + bvecs[i]
                return carry

            lax.fori_loop(0, CH, row_body, 0)

        start_gather(0, 0)
        start_gather(1, 1)

        def macro_body(i, carry):
            for b in range(NB):
                s_ = NB * i + b
                bp = (b + 2) % NB

                @pl.when(s_ >= 2)
                def _():
                    wait_out(bp)

                @pl.when(s_ + 2 < n_chunks)
                def _():
                    start_gather(bp, s_ + 2)

                wait_gather(b)
                compute(rows.at[b])
                start_out(b, s_)
            return carry

        lax.fori_loop(0, n_chunks // NB, macro_body, 0)
        wait_out((n_chunks - 2) % NB)
        wait_out((n_chunks - 1) % NB)

    return k


def kernel(x, table, gamma, beta):
    b, h = x.shape
    n_rows = b * h
    xf = x.reshape((n_rows,)).astype(jnp.int32)
    gb = jnp.stack([gamma, beta]).astype(jnp.float32)
    out = _make_kernel(n_rows)(xf, table, gb)
    return out.reshape((b, h, D))


# speedup vs baseline: 1.9736x; 1.9736x over previous
import functools

import jax
import jax.numpy as jnp
from jax import lax
from jax.experimental import pallas as pl
from jax.experimental.pallas import tpu as pltpu
from jax.experimental.pallas import tpu_sc as plsc

D = 64
EPS = 1e-5
L = 16
NC = 2
NS = 16
NW = NC * NS
CH = 128
NB = 4


def _rsqrt_vec(x):
    i = lax.bitcast_convert_type(x, jnp.int32)
    i = jnp.int32(0x5F375A86) - lax.shift_right_logical(i, 1)
    y = lax.bitcast_convert_type(i, jnp.float32)
    for _ in range(3):
        y = y * (jnp.float32(1.5) - jnp.float32(0.5) * x * y * y)
    return y


def _make_kernel(n_rows):
    rows_per_worker = n_rows // NW
    n_chunks = rows_per_worker // CH
    mesh = plsc.VectorSubcoreMesh(
        core_axis_name="c", subcore_axis_name="s", num_cores=NC, num_subcores=NS
    )

    @functools.partial(
        pl.kernel,
        out_type=jax.ShapeDtypeStruct((n_rows, D), jnp.float32),
        mesh=mesh,
        scratch_types=[
            pltpu.VMEM((rows_per_worker,), jnp.int32),
            pltpu.VMEM((NB, CH, D), jnp.float32),
            pltpu.VMEM((2, D), jnp.float32),
        ]
        + [pltpu.SemaphoreType.DMA] * (2 * NB),
        compiler_params=pltpu.CompilerParams(
            use_tc_tiling_on_sc=False, needs_layout_passes=False
        ),
    )
    def k(x_hbm, table_hbm, gb_hbm, out_hbm, idx_all, rows, gb_v, *sems):
        gsem = sems[:NB]
        osem = sems[NB:]
        wid = lax.axis_index("s") * NC + lax.axis_index("c")
        base = wid * rows_per_worker
        pltpu.sync_copy(x_hbm.at[pl.ds(base, rows_per_worker)], idx_all)
        pltpu.sync_copy(gb_hbm, gb_v)

        lane = lax.iota(jnp.int32, L)
        zvec = lax.bitwise_xor(lane, lane)
        gvecs = [gb_v[0, pl.ds(i * L, L)] for i in range(D // L)]
        bvecs = [gb_v[1, pl.ds(i * L, L)] for i in range(D // L)]

        def start_gather(b, j):
            pltpu.async_copy(
                table_hbm.at[idx_all.at[pl.ds(j * CH, CH)]], rows.at[b], gsem[b]
            )

        def wait_gather(b):
            pltpu.make_async_copy(
                table_hbm.at[pl.ds(0, CH)], rows.at[b], gsem[b]
            ).wait()

        def start_out(b, j):
            pltpu.async_copy(
                rows.at[b], out_hbm.at[pl.ds(base + j * CH, CH)], osem[b]
            )

        def wait_out(b):
            pltpu.make_async_copy(
                rows.at[b], out_hbm.at[pl.ds(0, CH)], osem[b]
            ).wait()

        def compute(rows_ref):
            def grp_body(g2, carry):
                row_ids = g2 * L + lane
                fzero = lax.convert_element_type(zvec, jnp.float32)
                acc = [fzero] * 4
                acc2 = [fzero] * 4
                for c in range(D):
                    col = lax.bitwise_and(lane + c, jnp.int32(D - 1))
                    v = plsc.load_gather(rows_ref, [row_ids, col])
                    r = c % 4
                    acc[r] = acc[r] + v
                    acc2[r] = acc2[r] + v * v
                s = (acc[0] + acc[1]) + (acc[2] + acc[3])
                q = (acc2[0] + acc2[1]) + (acc2[2] + acc2[3])
                mean = s * jnp.float32(1.0 / D)
                var = q * jnp.float32(1.0 / D) - mean * mean
                kk = _rsqrt_vec(var + jnp.float32(EPS))
                off = mean * kk
                for c in range(D):
                    col = lax.bitwise_and(lane + c, jnp.int32(D - 1))
                    v = plsc.load_gather(rows_ref, [row_ids, col])
                    plsc.store_scatter(rows_ref, [row_ids, col], v * kk - off)
                return carry

            lax.fori_loop(0, CH // L, grp_body, 0)

            def row_body(r, carry):
                for i in range(D // L):
                    sl = pl.ds(i * L, L)
                    rows_ref[r, sl] = rows_ref[r, sl] * gvecs[i] + bvecs[i]
                return carry

            lax.fori_loop(0, CH, row_body, 0)

        start_gather(0, 0)
        start_gather(1, 1)

        def macro_body(i, carry):
            for b in range(NB):
                s_ = NB * i + b
                bp = (b + 2) % NB

                @pl.when(s_ >= 2)
                def _():
                    wait_out(bp)

                @pl.when(s_ + 2 < n_chunks)
                def _():
                    start_gather(bp, s_ + 2)

                wait_gather(b)
                compute(rows.at[b])
                start_out(b, s_)
            return carry

        lax.fori_loop(0, n_chunks // NB, macro_body, 0)
        wait_out((n_chunks - 2) % NB)
        wait_out((n_chunks - 1) % NB)

    return k


def kernel(x, table, gamma, beta):
    b, h = x.shape
    n_rows = b * h
    xf = x.reshape((n_rows,)).astype(jnp.int32)
    gb = jnp.stack([gamma, beta]).astype(jnp.float32)
    out = _make_kernel(n_rows)(xf, table, gb)
    return out.reshape((b, h, D))
